# Initial kernel scaffold; baseline (speedup 1.0000x reference)
import functools

import jax
import jax.numpy as jnp
from jax import lax
from jax.experimental import pallas as pl
from jax.experimental.pallas import tpu as pltpu
from jax.experimental.pallas import tpu_sc as plsc

N = 10000
D = 128
E = 320000

NC = 2
NS = 16
NW = NC * NS
L = 16

K = 128
EPW = ((E // NW + K - 1) // K) * K
NCHUNK = EPW // K
EPAD = EPW * NW
RPT = N // NS
ZR = 125


def _sc_body(col_hbm, row_hbm, w_hbm, ann_hbm, part_hbm,
             col_v, row_v, w_v, rows_v, sem):
    cid = lax.axis_index("c")
    sid = lax.axis_index("s")
    wid = sid * NC + cid

    def acc_scope(acc):
        def zero_rows(e, _):
            for j in range(D // L):
                rows_v[e, pl.ds(j * L, L)] = jnp.zeros((L,), jnp.float32)
            return 0
        lax.fori_loop(0, ZR, zero_rows, 0)
        for i in range(RPT // ZR):
            pltpu.sync_copy(rows_v.at[pl.ds(0, ZR)],
                            acc.at[pl.ds(sid * RPT + i * ZR, ZR)])
        plsc.subcore_barrier()

        def chunk_body(c, _):
            base = pl.multiple_of(wid * EPW + c * K, K)
            pltpu.sync_copy(col_hbm.at[pl.ds(base, K)], col_v)
            pltpu.sync_copy(row_hbm.at[pl.ds(base, K)], row_v)
            pltpu.sync_copy(w_hbm.at[pl.ds(base, K)], w_v)
            pltpu.async_copy(ann_hbm.at[col_v], rows_v, sem).wait()

            def mul_body(e, _):
                ws = jnp.full((L,), w_v[e], jnp.float32)
                for j in range(D // L):
                    rows_v[e, pl.ds(j * L, L)] = rows_v[e, pl.ds(j * L, L)] * ws
                return 0
            lax.fori_loop(0, K, mul_body, 0)

            pltpu.sync_copy(rows_v, acc.at[row_v], add=True)
            return 0
        lax.fori_loop(0, NCHUNK, chunk_body, 0)
        plsc.subcore_barrier()

        for i in range(RPT // ZR):
            r0 = sid * RPT + i * ZR
            pltpu.sync_copy(acc.at[pl.ds(r0, ZR)],
                            part_hbm.at[pl.ds(cid * N + r0, ZR)])

    pl.run_scoped(acc_scope, pltpu.VMEM_SHARED((N, D), jnp.float32))


_sc_call = pl.kernel(
    _sc_body,
    out_type=jax.ShapeDtypeStruct((NC * N, D), jnp.float32),
    mesh=plsc.VectorSubcoreMesh(core_axis_name="c", subcore_axis_name="s"),
    scratch_types=[
        pltpu.VMEM((K,), jnp.int32),
        pltpu.VMEM((K,), jnp.int32),
        pltpu.VMEM((K,), jnp.float32),
        pltpu.VMEM((K, D), jnp.float32),
        pltpu.SemaphoreType.DMA,
    ],
)


def _add_body(a_ref, b_ref, o_ref):
    o_ref[...] = a_ref[...] + b_ref[...]


_BM = 2000


def _add_partials(part):
    return pl.pallas_call(
        _add_body,
        grid=(N // _BM,),
        in_specs=[
            pl.BlockSpec((_BM, D), lambda i: (i, 0)),
            pl.BlockSpec((_BM, D), lambda i: (i + N // _BM, 0)),
        ],
        out_specs=pl.BlockSpec((_BM, D), lambda i: (i, 0)),
        out_shape=jax.ShapeDtypeStruct((N, D), jnp.float32),
    )(part, part)


@jax.jit
def kernel(edge_index, edge_weight, annotations):
    pad = EPAD - E
    col = jnp.pad(edge_index[1], (0, pad))
    row = jnp.pad(edge_index[0], (0, pad))
    w = jnp.pad(edge_weight, (0, pad))
    part = _sc_call(col, row, w, annotations)
    return _add_partials(part)


# speedup vs baseline: 3.9762x; 3.9762x over previous
import functools

import jax
import jax.numpy as jnp
from jax import lax
from jax.experimental import pallas as pl
from jax.experimental.pallas import tpu as pltpu
from jax.experimental.pallas import tpu_sc as plsc

N = 10000
D = 128
E = 320000

NC = 2
NS = 16
NW = NC * NS
L = 16

K = 128
EPW = ((E // NW + K - 1) // K) * K
NCHUNK = EPW // K
EPAD = EPW * NW
NACC = 10112
ZPT = NACC // NS
DPT = 624


def _sc_body(col_hbm, row_hbm, w_hbm, ann_hbm, part_hbm,
             col_v, row_v, w_v, rows_v, acc, sem):
    cid = lax.axis_index("c")
    sid = lax.axis_index("s")
    wid = sid * NC + cid

    if True:
        def zero_rows(e, _):
            for j in range(D // L):
                rows_v[e, pl.ds(j * L, L)] = jnp.zeros((L,), jnp.float32)
            return 0
        lax.fori_loop(0, K, zero_rows, 0)
        zbase = sid * ZPT
        for i in range(4):
            pltpu.sync_copy(rows_v.at[pl.ds(0, K)],
                            acc.at[pl.ds(zbase + i * K, K)])
        pltpu.sync_copy(rows_v.at[pl.ds(0, ZPT - 4 * K)],
                        acc.at[pl.ds(zbase + 4 * K, ZPT - 4 * K)])
        plsc.subcore_barrier()

        def chunk_body(c, _):
            base = pl.multiple_of(wid * EPW + c * K, K)
            pltpu.sync_copy(col_hbm.at[pl.ds(base, K)], col_v)
            pltpu.sync_copy(row_hbm.at[pl.ds(base, K)], row_v)
            pltpu.sync_copy(w_hbm.at[pl.ds(base, K)], w_v)
            pltpu.async_copy(ann_hbm.at[col_v], rows_v, sem).wait()

            def mul_body(g, _):
                wv = w_v[pl.ds(g * L, L)]
                for l in range(L):
                    e = g * L + l
                    ws = jnp.full((L,), wv[l], jnp.float32)
                    for j in range(D // L):
                        rows_v[e, pl.ds(j * L, L)] = (
                            rows_v[e, pl.ds(j * L, L)] * ws)
                return 0
            lax.fori_loop(0, K // L, mul_body, 0)

            pltpu.sync_copy(rows_v, acc.at[row_v], add=True)
            return 0
        lax.fori_loop(0, NCHUNK, chunk_body, 0)
        plsc.subcore_barrier()

        dbase = DPT * sid + 8 * jnp.minimum(sid, 2)
        pltpu.sync_copy(acc.at[pl.ds(dbase, DPT)],
                        part_hbm.at[pl.ds(cid * N + dbase, DPT)])

        @pl.when(sid < 2)
        def _():
            pltpu.sync_copy(acc.at[pl.ds(dbase + DPT, 8)],
                            part_hbm.at[pl.ds(cid * N + dbase + DPT, 8)])


@functools.cache
def _sc_call():
    return pl.kernel(
        _sc_body,
        out_type=jax.ShapeDtypeStruct((NC * N, D), jnp.float32),
        mesh=plsc.VectorSubcoreMesh(core_axis_name="c", subcore_axis_name="s",
                                    num_cores=NC, num_subcores=NS),
        scratch_types=[
            pltpu.VMEM((K,), jnp.int32),
            pltpu.VMEM((K,), jnp.int32),
            pltpu.VMEM((K,), jnp.float32),
            pltpu.VMEM((K, D), jnp.float32),
            pltpu.VMEM_SHARED((NACC, D), jnp.float32),
            pltpu.SemaphoreType.DMA,
        ],
    )


def _add_body(a_ref, b_ref, o_ref):
    o_ref[...] = a_ref[...] + b_ref[...]


_BM = 2000


def _add_partials(part):
    return pl.pallas_call(
        _add_body,
        grid=(N // _BM,),
        in_specs=[
            pl.BlockSpec((_BM, D), lambda i: (i, 0)),
            pl.BlockSpec((_BM, D), lambda i: (i + N // _BM, 0)),
        ],
        out_specs=pl.BlockSpec((_BM, D), lambda i: (i, 0)),
        out_shape=jax.ShapeDtypeStruct((N, D), jnp.float32),
    )(part, part)


@jax.jit
def kernel(edge_index, edge_weight, annotations):
    pad = EPAD - E
    col = jnp.pad(edge_index[1], (0, pad))
    row = jnp.pad(edge_index[0], (0, pad))
    w = jnp.pad(edge_weight, (0, pad))
    part = _sc_call()(col, row, w, annotations)
    return _add_partials(part)
